# baseline (speedup 1.0000x reference)
import functools

import jax
import jax.numpy as jnp
from jax import lax
from jax.experimental import pallas as pl
from jax.experimental.pallas import tpu as pltpu
from jax.experimental.pallas import tpu_sc as plsc

NUM_FIELDS = 26
VOCAB = 100000
EMB_DIM = 32
BATCH = 16384

N_TROWS = NUM_FIELDS * EMB_DIM
NW = 32
LANES = 16
N_SPLIT = 2
SPLIT_ROWS = N_TROWS // N_SPLIT
R_PER_W = SPLIT_ROWS // NW
OUT_CHUNK = 8192
N_OCHUNK = BATCH // OUT_CHUNK

_mesh = plsc.VectorSubcoreMesh(core_axis_name="c", subcore_axis_name="s")


def _make_half_kernel(r_off):
    @functools.partial(
        pl.kernel,
        out_type=jax.ShapeDtypeStruct((SPLIT_ROWS, BATCH), jnp.float32),
        mesh=_mesh,
        scratch_types=[
            pltpu.VMEM((VOCAB,), jnp.float32),
            pltpu.VMEM((BATCH,), jnp.int32),
            pltpu.VMEM((OUT_CHUNK,), jnp.float32),
            pltpu.SemaphoreType.DMA,
        ],
        compiler_params=pltpu.CompilerParams(
            use_tc_tiling_on_sc=False, needs_layout_passes=False
        ),
    )
    def _half_kernel(tt_hbm, xt_hbm, out_hbm, row_v, xb_v, ob_v, sem):
        wid = lax.axis_index("s") * 2 + lax.axis_index("c")
        r_base = wid * R_PER_W

        def _row_body(i_r, f_prev):
            r = r_base + i_r
            f = (r + r_off) // EMB_DIM

            @pl.when(f != f_prev)
            def _load_x():
                pltpu.sync_copy(xt_hbm.at[f], xb_v)

            pltpu.sync_copy(tt_hbm.at[r], row_v)

            def _ochunk_body(oc, carry2):
                @plsc.parallel_loop(0, OUT_CHUNK // LANES, unroll=8)
                def _vec_body(i):
                    xv = xb_v[pl.ds(oc * OUT_CHUNK + i * LANES, LANES)]
                    ob_v[pl.ds(i * LANES, LANES)] = plsc.load_gather(row_v, [xv])

                pltpu.sync_copy(
                    ob_v, out_hbm.at[r, pl.ds(oc * OUT_CHUNK, OUT_CHUNK)]
                )
                return carry2

            lax.fori_loop(0, N_OCHUNK, _ochunk_body, 0)
            return f

        lax.fori_loop(0, R_PER_W, _row_body, jnp.int32(-1))

    return _half_kernel


_half_kernels = [_make_half_kernel(s * SPLIT_ROWS) for s in range(N_SPLIT)]


def kernel(x, tables):
    tt = tables.transpose(0, 2, 1).reshape(N_TROWS, VOCAB)
    xt = x.astype(jnp.int32).T
    parts = [
        _half_kernels[s](tt[s * SPLIT_ROWS:(s + 1) * SPLIT_ROWS], xt)
        for s in range(N_SPLIT)
    ]
    out_t = jnp.concatenate(parts, axis=0)
    return out_t.T.reshape(BATCH, NUM_FIELDS * EMB_DIM)


# speedup vs baseline: 1.3864x; 1.3864x over previous
import functools

import jax
import jax.numpy as jnp
from jax import lax
from jax.experimental import pallas as pl
from jax.experimental.pallas import tpu as pltpu
from jax.experimental.pallas import tpu_sc as plsc

NUM_FIELDS = 26
VOCAB = 100000
EMB_DIM = 32
BATCH = 16384

N_TROWS = NUM_FIELDS * EMB_DIM
NW = 32
LANES = 16
R_PER_W = N_TROWS // NW
OUT_CHUNK = 8192
N_OCHUNK = BATCH // OUT_CHUNK

_mesh = plsc.VectorSubcoreMesh(core_axis_name="c", subcore_axis_name="s")


@functools.partial(
    pl.kernel,
    out_type=jax.ShapeDtypeStruct(
        (N_TROWS // 8, BATCH // 128, 8, 128), jnp.float32
    ),
    mesh=_mesh,
    scratch_types=[
        pltpu.VMEM((VOCAB,), jnp.float32),
        pltpu.VMEM((BATCH,), jnp.int32),
        pltpu.VMEM((OUT_CHUNK // 128, 128), jnp.float32),
        pltpu.SemaphoreType.DMA,
    ],
    compiler_params=pltpu.CompilerParams(
        use_tc_tiling_on_sc=False, needs_layout_passes=False
    ),
)
def _gather_kernel(tt_hbm, xt_hbm, out_hbm, row_v, xb_v, ob_v, sem):
    wid = lax.axis_index("s") * 2 + lax.axis_index("c")
    r_base = wid * R_PER_W

    def _row_body(i_r, f_prev):
        r = r_base + i_r
        f = r // EMB_DIM

        @pl.when(f != f_prev)
        def _load_x():
            pltpu.sync_copy(xt_hbm.at[f], xb_v)

        pltpu.sync_copy(tt_hbm.at[r], row_v)

        g = r // 8
        rm = r % 8

        def _ochunk_body(oc, carry2):
            @plsc.parallel_loop(0, OUT_CHUNK // LANES, unroll=8)
            def _vec_body(i):
                xv = xb_v[pl.ds(oc * OUT_CHUNK + i * LANES, LANES)]
                cc = i // (128 // LANES)
                k = i % (128 // LANES)
                ob_v[cc, pl.ds(k * LANES, LANES)] = plsc.load_gather(row_v, [xv])

            pltpu.sync_copy(
                ob_v,
                out_hbm.at[g, pl.ds(oc * (OUT_CHUNK // 128), OUT_CHUNK // 128), rm],
            )
            return carry2

        lax.fori_loop(0, N_OCHUNK, _ochunk_body, 0)
        return f

    lax.fori_loop(0, R_PER_W, _row_body, jnp.int32(-1))


def kernel(x, tables):
    tt = tables.transpose(0, 2, 1).reshape(N_TROWS, VOCAB)
    xt = x.astype(jnp.int32).T
    out4 = _gather_kernel(tt, xt)
    return out4.transpose(1, 3, 0, 2).reshape(BATCH, NUM_FIELDS * EMB_DIM)
